# baseline (speedup 1.0000x reference)
import jax
import jax.numpy as jnp
from jax import lax
from jax.experimental import pallas as pl
from jax.experimental.pallas import tpu as pltpu
from jax.experimental.pallas import tpu_sc as plsc

_B, _S, _D = 1, 2048, 768
_E, _T, _K, _H = 16, 5, 2, 768
_N = _B * _S
_NPAIR = _N * _K
_NW = 32
_PPW = _NPAIR // _NW
_R = 256
_NT = _NPAIR // _R + _E
_NP = _NT * _R
_MW = 128


def _cv2(v, n):
    mu = jnp.sum(v, keepdims=True) / n
    var = jnp.sum((v - mu) ** 2, keepdims=True) / (n - 1)
    return var / (mu * mu + 1e-10)


def _cv2_col(v):
    mu = jnp.sum(v, axis=0, keepdims=True) / _E
    var = jnp.sum((v - mu) ** 2, axis=0, keepdims=True) / (_E - 1)
    return var / (mu * mu + 1e-10)


def _gate_body(x_ref, gwt_ref, gb_ref, gws_ref, gbs_ref, ti_ref,
               pos_ref, meta_ref, loss_ref, imp_ref, te_ref, nv_ref):
    logits = jax.lax.dot_general(
        gwt_ref[...], x_ref[...], (((1,), (1,)), ((), ())),
        preferred_element_type=jnp.float32) + gb_ref[...]
    ti = ti_ref[...]
    io0 = jax.lax.broadcasted_iota(jnp.int32, (_E, _N), 0)
    loss = jnp.zeros((1, 1), jnp.float32)
    i1r = jnp.zeros((1, _N), jnp.int32)
    i2r = jnp.zeros((1, _N), jnp.int32)
    impc = jnp.zeros((_E, 1), jnp.float32)
    for t in range(_T):
        lt = logits[t * _E:(t + 1) * _E, :]
        m1 = jnp.max(lt, axis=0, keepdims=True)
        i1 = jnp.min(jnp.where(lt == m1, io0, _E), axis=0, keepdims=True)
        lt2 = jnp.where(io0 == i1, -jnp.inf, lt)
        m2 = jnp.max(lt2, axis=0, keepdims=True)
        i2 = jnp.min(jnp.where(lt2 == m2, io0, _E), axis=0, keepdims=True)
        wa = jax.nn.sigmoid(m1 - m2)
        wb = jax.nn.sigmoid(m2 - m1)
        gt = jnp.where(io0 == i1, wa, 0.0) + jnp.where(io0 == i2, wb, 0.0)
        imp = jnp.sum(gt, axis=1, keepdims=True)
        ld = jnp.sum((gt > 0).astype(jnp.float32), axis=1, keepdims=True)
        loss = loss + _cv2_col(imp) + _cv2_col(ld)
        sel = ti == t
        i1r = jnp.where(sel, i1, i1r)
        i2r = jnp.where(sel, i2, i2r)
        impc = jnp.where(sel, imp, impc)
    loss_ref[...] = loss
    imp_ref[...] = impc

    ioc = jax.lax.broadcasted_iota(jnp.int32, (_N, _E), 1)
    ltc = jax.lax.dot_general(
        x_ref[...], gws_ref[...], (((1,), (1,)), ((), ())),
        preferred_element_type=jnp.float32) + gbs_ref[...]
    m1c = jnp.max(ltc, axis=1, keepdims=True)
    i1c = jnp.min(jnp.where(ltc == m1c, ioc, _E), axis=1, keepdims=True)
    ltc2 = jnp.where(ioc == i1c, -jnp.inf, ltc)
    m2c = jnp.max(ltc2, axis=1, keepdims=True)
    wac = jax.nn.sigmoid(m1c - m2c)
    wbc = jax.nn.sigmoid(m2c - m1c)
    gpair = jnp.concatenate([wac, wbc], axis=0)
    meta_ref[...] = jnp.concatenate(
        [gpair, jnp.zeros((_NPAIR, _MW - 1), jnp.float32)], axis=1)

    iop = jax.lax.broadcasted_iota(jnp.int32, (_E, _NPAIR), 0)
    eids = jnp.concatenate([i1r, i2r], axis=1)
    onehot = (iop == eids).astype(jnp.int32)
    incl = onehot
    k = 1
    while k < _NPAIR:
        incl = incl + jnp.concatenate(
            [jnp.zeros((_E, k), jnp.int32), incl[:, :_NPAIR - k]], axis=1)
        k *= 2
    excl = incl - onehot
    cnt = incl[:, _NPAIR - 1:_NPAIR]
    region = ((cnt + _R - 1) // _R) * _R
    ioc16 = jax.lax.broadcasted_iota(jnp.int32, (_E, 1), 0)
    io_t = jax.lax.broadcasted_iota(jnp.int32, (1, _NT), 1)

    def row(vec, e):
        return jnp.sum(jnp.where(ioc16 == e, vec, 0), axis=0, keepdims=True)

    starts = jnp.zeros((_E, 1), jnp.int32)
    acc = jnp.zeros((1, 1), jnp.int32)
    for e in range(_E):
        starts = starts + jnp.where(ioc16 == e, acc, 0)
        acc = acc + row(region, e)
    posr = jnp.sum((excl + starts) * onehot, axis=0, keepdims=True)
    pos_ref[...] = posr.reshape(_NW, _PPW)

    writ_end = starts + cnt
    tiles_incl = (starts + region) // _R
    te = jnp.zeros((1, _NT), jnp.int32)
    for e in range(_E):
        te = te + (io_t >= row(tiles_incl, e)).astype(jnp.int32)
    te = jnp.minimum(te, _E - 1)
    we_t = jnp.zeros((1, _NT), jnp.int32)
    for e in range(_E):
        we_t = we_t + jnp.where(te == e, 1, 0) * row(writ_end, e)
    nv_ref[...] = jnp.clip(we_t - io_t * _R, 0, _R).reshape(_NT)
    te_ref[...] = te.reshape(_NT)


def _sc_body(pos_hbm, meta_hbm, x_hbm, xs_hbm, ms_hbm,
             pos_v, meta_v, rows_v, sem, sem2):
    c = lax.axis_index("c")
    s = lax.axis_index("s")
    wid = s * 2 + c
    tok_base = pl.multiple_of((wid * _PPW) % _N, _PPW)
    pltpu.sync_copy(pos_hbm.at[wid], pos_v)
    pltpu.sync_copy(meta_hbm.at[pl.ds(wid * _PPW, _PPW)], meta_v)
    pltpu.sync_copy(x_hbm.at[pl.ds(tok_base, _PPW)], rows_v)
    pltpu.async_copy(rows_v, xs_hbm.at[pos_v], sem).wait()
    pltpu.async_copy(meta_v, ms_hbm.at[pos_v], sem2).wait()


def _sc_dispatch(pos2d, meta, x):
    fn = pl.kernel(
        _sc_body,
        out_type=[jax.ShapeDtypeStruct((_NP, _D), jnp.float32),
                  jax.ShapeDtypeStruct((_NP, _MW), jnp.float32)],
        mesh=plsc.VectorSubcoreMesh(core_axis_name="c", subcore_axis_name="s"),
        scratch_types=[
            pltpu.VMEM((_PPW,), jnp.int32),
            pltpu.VMEM((_PPW, _MW), jnp.float32),
            pltpu.VMEM((_PPW, _D), jnp.float32),
            pltpu.SemaphoreType.DMA,
            pltpu.SemaphoreType.DMA,
        ],
    )
    return fn(pos2d, meta, x)


def _mm_body(te_ref, nv_ref, xs_ref, ms_ref, w1_ref, b1_ref, w2_ref, b2_ref,
             imp_ref, loss_ref, lng_ref, lnb_ref, hw_ref, hb_ref, ty_ref,
             scores_ref, tot_ref, ploss_ref, ysum_ref, vacc_ref):
    s = pl.program_id(0)

    @pl.when(s == 0)
    def _init():
        ysum_ref[...] = jnp.zeros((1, _H), jnp.float32)
        vacc_ref[...] = jnp.zeros((1, _H), jnp.float32)

    nv = nv_ref[s]

    @pl.when(nv > 0)
    def _tile():
        rows = xs_ref[...]
        h = jax.nn.relu(
            jax.lax.dot_general(rows, w1_ref[0], (((1,), (1,)), ((), ())),
                                preferred_element_type=jnp.float32)
            + b1_ref[0])
        valid = jax.lax.broadcasted_iota(jnp.int32, (_R, 1), 0) < nv
        g = jnp.where(valid, ms_ref[:, 0:1], 0.0)
        h = jnp.where(valid, h, 0.0)
        hg = h * g
        k = _R // 2
        while k >= 8:
            hg = hg[:k] + hg[k:]
            k //= 2
        vacc_ref[...] += jnp.sum(hg, axis=0, keepdims=True)
        nxt = jnp.minimum(s + 1, _NT - 1)
        flush = ((s == _NT - 1) | (te_ref[nxt] != te_ref[s])
                 | (nv_ref[nxt] == 0))

        @pl.when(flush)
        def _flush():
            ysum_ref[...] += jax.lax.dot_general(
                vacc_ref[...], w2_ref[0], (((1,), (1,)), ((), ())),
                precision=jax.lax.Precision.HIGHEST,
                preferred_element_type=jnp.float32)
            vacc_ref[...] = jnp.zeros((1, _H), jnp.float32)

    @pl.when(s == _NT - 1)
    def _final():
        mm = ysum_ref[...] + jax.lax.dot_general(
            imp_ref[...], b2_ref[...], (((0,), (0,)), ((), ())),
            precision=jax.lax.Precision.HIGHEST,
            preferred_element_type=jnp.float32)
        mu = jnp.sum(mm, keepdims=True) / _H
        var = jnp.sum((mm - mu) ** 2, keepdims=True) / _H
        fin = (mm - mu) / jnp.sqrt(var + 1e-5) * lng_ref[...] + lnb_ref[...]
        out = jnp.sum(fin * hw_ref[...], keepdims=True) + hb_ref[...]
        sc = jax.nn.sigmoid(out)
        scores_ref[...] = sc
        tot_ref[...] = loss_ref[...] * 0.01
        ploss_ref[...] = (sc - ty_ref[...]) ** 2


@jax.jit
def kernel(mm_embed, task_index, true_y, gate_W, gate_b, exp_W1, exp_b1,
           exp_W2, exp_b2, ln_g, ln_b, head_W, head_b):
    x = mm_embed.reshape(_N, _D)
    gwt = gate_W.reshape(_T * _E, _D)
    gb = gate_b.reshape(_T * _E, 1)
    ti = task_index.reshape(1, 1)
    ty = true_y.reshape(1, 1)

    full = lambda s: pl.BlockSpec(s, lambda *_: (0,) * len(s))

    gws = jax.lax.dynamic_index_in_dim(
        gate_W, task_index[0], axis=0, keepdims=False).reshape(_E, _D)
    gbs = jax.lax.dynamic_index_in_dim(
        gate_b, task_index[0], axis=0, keepdims=False).reshape(1, _E)

    pos2d, meta, loss, imp, te, nv = pl.pallas_call(
        _gate_body,
        in_specs=[full((_N, _D)), full((_T * _E, _D)), full((_T * _E, 1)),
                  full((_E, _D)), full((1, _E)), full((1, 1))],
        out_specs=[full((_NW, _PPW)), full((_NPAIR, _MW)), full((1, 1)),
                   full((_E, 1)), full((_NT,)), full((_NT,))],
        out_shape=[
            jax.ShapeDtypeStruct((_NW, _PPW), jnp.int32),
            jax.ShapeDtypeStruct((_NPAIR, _MW), jnp.float32),
            jax.ShapeDtypeStruct((1, 1), jnp.float32),
            jax.ShapeDtypeStruct((_E, 1), jnp.float32),
            jax.ShapeDtypeStruct((_NT,), jnp.int32),
            jax.ShapeDtypeStruct((_NT,), jnp.int32),
        ],
    )(x, gwt, gb, gws, gbs, ti)

    xs, ms = _sc_dispatch(pos2d, meta, x)

    def xs_map(s, te_r, nv_r):
        return (jnp.where(nv_r[s] > 0, s, 0), 0)

    def w_map(s, te_r, nv_r):
        return (te_r[s], 0, 0)

    fullp = lambda s: pl.BlockSpec(s, lambda i, te_r, nv_r: (0,) * len(s))

    scores, tot, ploss = pl.pallas_call(
        _mm_body,
        grid_spec=pltpu.PrefetchScalarGridSpec(
            num_scalar_prefetch=2,
            grid=(_NT,),
            in_specs=[
                pl.BlockSpec((_R, _D), xs_map),
                pl.BlockSpec((_R, _MW), xs_map),
                pl.BlockSpec((1, _H, _D), w_map),
                pl.BlockSpec((1, 1, _H), w_map),
                pl.BlockSpec((1, _D, _H), w_map),
                fullp((_E, _D)),
                fullp((_E, 1)),
                fullp((1, 1)),
                fullp((1, _H)),
                fullp((1, _H)),
                fullp((1, _H)),
                fullp((1, 1)),
                fullp((1, 1)),
            ],
            out_specs=[fullp((1, 1))] * 3,
            scratch_shapes=[pltpu.VMEM((1, _H), jnp.float32),
                            pltpu.VMEM((1, _H), jnp.float32)],
        ),
        out_shape=[jax.ShapeDtypeStruct((1, 1), jnp.float32)] * 3,
    )(te, nv,
      xs, ms, exp_W1, exp_b1.reshape(_E, 1, _H), exp_W2, exp_b2,
      imp, loss, ln_g.reshape(1, _H), ln_b.reshape(1, _H),
      head_W.reshape(1, _H), head_b.reshape(1, 1), ty)

    return (scores.astype(jnp.float32),
            tot.reshape(()).astype(jnp.float32),
            ploss.reshape(()).astype(jnp.float32))


# speedup vs baseline: 1.1227x; 1.1227x over previous
import jax
import jax.numpy as jnp
from jax import lax
from jax.experimental import pallas as pl
from jax.experimental.pallas import tpu as pltpu
from jax.experimental.pallas import tpu_sc as plsc

_B, _S, _D = 1, 2048, 768
_E, _T, _K, _H = 16, 5, 2, 768
_N = _B * _S
_NPAIR = _N * _K
_NW = 32
_PPW = _NPAIR // _NW
_R = 256
_NT = _NPAIR // _R + _E
_NP = _NT * _R
_MW = 128


def _cv2(v, n):
    mu = jnp.sum(v, keepdims=True) / n
    var = jnp.sum((v - mu) ** 2, keepdims=True) / (n - 1)
    return var / (mu * mu + 1e-10)


def _cv2_col(v):
    mu = jnp.sum(v, axis=0, keepdims=True) / _E
    var = jnp.sum((v - mu) ** 2, axis=0, keepdims=True) / (_E - 1)
    return var / (mu * mu + 1e-10)


def _gate_body(x_ref, gwt_ref, gb_ref, gws_ref, gbs_ref, ti_ref,
               pos_ref, meta_ref, loss_ref, imp_ref, te_ref, nv_ref):
    logits = jax.lax.dot_general(
        gwt_ref[...], x_ref[...], (((1,), (1,)), ((), ())),
        preferred_element_type=jnp.float32) + gb_ref[...]
    ti = ti_ref[...]
    io0 = jax.lax.broadcasted_iota(jnp.int32, (_E, _N), 0)
    loss = jnp.zeros((1, 1), jnp.float32)
    i1r = jnp.zeros((1, _N), jnp.int32)
    i2r = jnp.zeros((1, _N), jnp.int32)
    impc = jnp.zeros((_E, 1), jnp.float32)
    for t in range(_T):
        lt = logits[t * _E:(t + 1) * _E, :]
        m1 = jnp.max(lt, axis=0, keepdims=True)
        i1 = jnp.min(jnp.where(lt == m1, io0, _E), axis=0, keepdims=True)
        lt2 = jnp.where(io0 == i1, -jnp.inf, lt)
        m2 = jnp.max(lt2, axis=0, keepdims=True)
        i2 = jnp.min(jnp.where(lt2 == m2, io0, _E), axis=0, keepdims=True)
        wa = jax.nn.sigmoid(m1 - m2)
        wb = jax.nn.sigmoid(m2 - m1)
        gt = jnp.where(io0 == i1, wa, 0.0) + jnp.where(io0 == i2, wb, 0.0)
        imp = jnp.sum(gt, axis=1, keepdims=True)
        ld = jnp.sum((gt > 0).astype(jnp.float32), axis=1, keepdims=True)
        loss = loss + _cv2_col(imp) + _cv2_col(ld)
        sel = ti == t
        i1r = jnp.where(sel, i1, i1r)
        i2r = jnp.where(sel, i2, i2r)
        impc = jnp.where(sel, imp, impc)
    loss_ref[...] = loss
    imp_ref[...] = impc

    ioc = jax.lax.broadcasted_iota(jnp.int32, (_N, _E), 1)
    ltc = jax.lax.dot_general(
        x_ref[...], gws_ref[...], (((1,), (1,)), ((), ())),
        preferred_element_type=jnp.float32) + gbs_ref[...]
    m1c = jnp.max(ltc, axis=1, keepdims=True)
    i1c = jnp.min(jnp.where(ltc == m1c, ioc, _E), axis=1, keepdims=True)
    ltc2 = jnp.where(ioc == i1c, -jnp.inf, ltc)
    m2c = jnp.max(ltc2, axis=1, keepdims=True)
    wac = jax.nn.sigmoid(m1c - m2c)
    wbc = jax.nn.sigmoid(m2c - m1c)
    gpair = jnp.concatenate([wac, wbc], axis=0)
    meta_ref[...] = jnp.concatenate(
        [gpair, jnp.zeros((_NPAIR, _MW - 1), jnp.float32)], axis=1)

    iop = jax.lax.broadcasted_iota(jnp.int32, (_E, _NPAIR), 0)
    eids = jnp.concatenate([i1r, i2r], axis=1)
    onehot = (iop == eids).astype(jnp.int32)
    incl = onehot
    k = 1
    while k < _NPAIR:
        incl = incl + jnp.concatenate(
            [jnp.zeros((_E, k), jnp.int32), incl[:, :_NPAIR - k]], axis=1)
        k *= 2
    excl = incl - onehot
    cnt = incl[:, _NPAIR - 1:_NPAIR]
    region = ((cnt + _R - 1) // _R) * _R
    ioc16 = jax.lax.broadcasted_iota(jnp.int32, (_E, 1), 0)
    io_t = jax.lax.broadcasted_iota(jnp.int32, (1, _NT), 1)

    def row(vec, e):
        return jnp.sum(jnp.where(ioc16 == e, vec, 0), axis=0, keepdims=True)

    starts = jnp.zeros((_E, 1), jnp.int32)
    acc = jnp.zeros((1, 1), jnp.int32)
    for e in range(_E):
        starts = starts + jnp.where(ioc16 == e, acc, 0)
        acc = acc + row(region, e)
    posr = jnp.sum((excl + starts) * onehot, axis=0, keepdims=True)
    pos_ref[...] = posr.reshape(_NW, _PPW)

    writ_end = starts + cnt
    tiles_incl = (starts + region) // _R
    te = jnp.zeros((1, _NT), jnp.int32)
    for e in range(_E):
        te = te + (io_t >= row(tiles_incl, e)).astype(jnp.int32)
    te = jnp.minimum(te, _E - 1)
    we_t = jnp.zeros((1, _NT), jnp.int32)
    for e in range(_E):
        we_t = we_t + jnp.where(te == e, 1, 0) * row(writ_end, e)
    nv_ref[...] = jnp.clip(we_t - io_t * _R, 0, _R).reshape(_NT)
    te_ref[...] = te.reshape(_NT)


def _sc_body(pos_hbm, meta_hbm, x_hbm, xs_hbm, ms_hbm,
             pos_v, meta_v, rows_v, sem, sem2):
    c = lax.axis_index("c")
    s = lax.axis_index("s")
    wid = s * 2 + c
    tok_base = pl.multiple_of((wid * _PPW) % _N, _PPW)
    pltpu.sync_copy(pos_hbm.at[wid], pos_v)
    pltpu.sync_copy(meta_hbm.at[pl.ds(wid * _PPW, _PPW)], meta_v)
    pltpu.sync_copy(x_hbm.at[pl.ds(tok_base, _PPW)], rows_v)
    pltpu.async_copy(rows_v, xs_hbm.at[pos_v], sem).wait()
    pltpu.async_copy(meta_v, ms_hbm.at[pos_v], sem2).wait()


def _sc_dispatch(pos2d, meta, x):
    fn = pl.kernel(
        _sc_body,
        out_type=[jax.ShapeDtypeStruct((_NP, _D), jnp.float32),
                  jax.ShapeDtypeStruct((_NP, _MW), jnp.float32)],
        mesh=plsc.VectorSubcoreMesh(core_axis_name="c", subcore_axis_name="s"),
        scratch_types=[
            pltpu.VMEM((_PPW,), jnp.int32),
            pltpu.VMEM((_PPW, _MW), jnp.float32),
            pltpu.VMEM((_PPW, _D), jnp.float32),
            pltpu.SemaphoreType.DMA,
            pltpu.SemaphoreType.DMA,
        ],
    )
    return fn(pos2d, meta, x)


def _mm_body(te_ref, nv_ref, xs_ref, ms_ref, w1_ref, b1_ref, w2_ref, b2_ref,
             imp_ref, loss_ref, lng_ref, lnb_ref, hw_ref, hb_ref, ty_ref,
             scores_ref, tot_ref, ploss_ref, ysum_ref, vacc_ref):
    s = pl.program_id(0)

    @pl.when(s == 0)
    def _init():
        ysum_ref[...] = jnp.zeros((1, _H), jnp.float32)
        vacc_ref[...] = jnp.zeros((1, _H), jnp.float32)

    nv = nv_ref[s]

    @pl.when(nv > 0)
    def _tile():
        rows = xs_ref[...]
        h = jax.nn.relu(
            jax.lax.dot_general(rows, w1_ref[0], (((1,), (1,)), ((), ())),
                                preferred_element_type=jnp.float32)
            + b1_ref[0])
        valid = jax.lax.broadcasted_iota(jnp.int32, (_R, 1), 0) < nv
        g = jnp.where(valid, ms_ref[:, 0:1], 0.0)
        h = jnp.where(valid, h, 0.0)
        hg = h * g
        k = _R // 2
        while k >= 8:
            hg = hg[:k] + hg[k:]
            k //= 2
        vacc_ref[...] += jnp.sum(hg, axis=0, keepdims=True)
        nxt = jnp.minimum(s + 1, _NT - 1)
        flush = ((s == _NT - 1) | (te_ref[nxt] != te_ref[s])
                 | (nv_ref[nxt] == 0))

        @pl.when(flush)
        def _flush():
            col = jnp.sum(w2_ref[0] * vacc_ref[...], axis=1, keepdims=True)
            ysum_ref[...] += col.reshape(1, _H)
            vacc_ref[...] = jnp.zeros((1, _H), jnp.float32)

    @pl.when(s == _NT - 1)
    def _final():
        mm = ysum_ref[...] + jax.lax.dot_general(
            imp_ref[...], b2_ref[...], (((0,), (0,)), ((), ())),
            precision=jax.lax.Precision.HIGHEST,
            preferred_element_type=jnp.float32)
        mu = jnp.sum(mm, keepdims=True) / _H
        var = jnp.sum((mm - mu) ** 2, keepdims=True) / _H
        fin = (mm - mu) / jnp.sqrt(var + 1e-5) * lng_ref[...] + lnb_ref[...]
        out = jnp.sum(fin * hw_ref[...], keepdims=True) + hb_ref[...]
        sc = jax.nn.sigmoid(out)
        scores_ref[...] = sc
        tot_ref[...] = loss_ref[...] * 0.01
        ploss_ref[...] = (sc - ty_ref[...]) ** 2


@jax.jit
def kernel(mm_embed, task_index, true_y, gate_W, gate_b, exp_W1, exp_b1,
           exp_W2, exp_b2, ln_g, ln_b, head_W, head_b):
    x = mm_embed.reshape(_N, _D)
    gwt = gate_W.reshape(_T * _E, _D)
    gb = gate_b.reshape(_T * _E, 1)
    ti = task_index.reshape(1, 1)
    ty = true_y.reshape(1, 1)

    full = lambda s: pl.BlockSpec(s, lambda *_: (0,) * len(s))

    gws = jax.lax.dynamic_index_in_dim(
        gate_W, task_index[0], axis=0, keepdims=False).reshape(_E, _D)
    gbs = jax.lax.dynamic_index_in_dim(
        gate_b, task_index[0], axis=0, keepdims=False).reshape(1, _E)

    pos2d, meta, loss, imp, te, nv = pl.pallas_call(
        _gate_body,
        in_specs=[full((_N, _D)), full((_T * _E, _D)), full((_T * _E, 1)),
                  full((_E, _D)), full((1, _E)), full((1, 1))],
        out_specs=[full((_NW, _PPW)), full((_NPAIR, _MW)), full((1, 1)),
                   full((_E, 1)), full((_NT,)), full((_NT,))],
        out_shape=[
            jax.ShapeDtypeStruct((_NW, _PPW), jnp.int32),
            jax.ShapeDtypeStruct((_NPAIR, _MW), jnp.float32),
            jax.ShapeDtypeStruct((1, 1), jnp.float32),
            jax.ShapeDtypeStruct((_E, 1), jnp.float32),
            jax.ShapeDtypeStruct((_NT,), jnp.int32),
            jax.ShapeDtypeStruct((_NT,), jnp.int32),
        ],
    )(x, gwt, gb, gws, gbs, ti)

    xs, ms = _sc_dispatch(pos2d, meta, x)

    def xs_map(s, te_r, nv_r):
        return (jnp.where(nv_r[s] > 0, s, 0), 0)

    def w_map(s, te_r, nv_r):
        return (te_r[s], 0, 0)

    fullp = lambda s: pl.BlockSpec(s, lambda i, te_r, nv_r: (0,) * len(s))

    scores, tot, ploss = pl.pallas_call(
        _mm_body,
        grid_spec=pltpu.PrefetchScalarGridSpec(
            num_scalar_prefetch=2,
            grid=(_NT,),
            in_specs=[
                pl.BlockSpec((_R, _D), xs_map),
                pl.BlockSpec((_R, _MW), xs_map),
                pl.BlockSpec((1, _H, _D), w_map),
                pl.BlockSpec((1, 1, _H), w_map),
                pl.BlockSpec((1, _D, _H), w_map),
                fullp((_E, _D)),
                fullp((_E, 1)),
                fullp((1, 1)),
                fullp((1, _H)),
                fullp((1, _H)),
                fullp((1, _H)),
                fullp((1, 1)),
                fullp((1, 1)),
            ],
            out_specs=[fullp((1, 1))] * 3,
            scratch_shapes=[pltpu.VMEM((1, _H), jnp.float32),
                            pltpu.VMEM((1, _H), jnp.float32)],
        ),
        out_shape=[jax.ShapeDtypeStruct((1, 1), jnp.float32)] * 3,
    )(te, nv,
      xs, ms, exp_W1, exp_b1.reshape(_E, 1, _H), exp_W2, exp_b2,
      imp, loss, ln_g.reshape(1, _H), ln_b.reshape(1, _H),
      head_W.reshape(1, _H), head_b.reshape(1, 1), ty)

    return (scores.astype(jnp.float32),
            tot.reshape(()).astype(jnp.float32),
            ploss.reshape(()).astype(jnp.float32))
